# baseline (speedup 1.0000x reference)
import functools

import jax
import jax.numpy as jnp
from jax import lax
from jax.experimental import pallas as pl
from jax.experimental.pallas import tpu as pltpu
from jax.experimental.pallas import tpu_sc as plsc

VOCAB = 1000
D_MODEL = 64
NUM_TOKENS = 1024 * 50

NUM_CORES = 2
NUM_SUBCORES = 16
NUM_WORKERS = NUM_CORES * NUM_SUBCORES
ROWS_PER_TILE = NUM_TOKENS // NUM_WORKERS
CHUNK = 64
NCHUNK = ROWS_PER_TILE // CHUNK


def _matmul_body(e_ref, p_ref, g_ref):
    g_ref[...] = lax.dot_general(
        e_ref[...], p_ref[...],
        dimension_numbers=(((1,), (1,)), ((), ())),
        preferred_element_type=jnp.float32,
    )


def _compute_g(embed_table, proj_W):
    return pl.pallas_call(
        _matmul_body,
        out_shape=jax.ShapeDtypeStruct((VOCAB, VOCAB), jnp.float32),
    )(embed_table, proj_W)


@functools.partial(
    pl.kernel,
    out_type=jax.ShapeDtypeStruct((NUM_TOKENS, VOCAB), jnp.float32),
    mesh=plsc.VectorSubcoreMesh(core_axis_name="c", subcore_axis_name="s"),
    scratch_types=[
        pltpu.VMEM((ROWS_PER_TILE,), jnp.int32),
        pltpu.VMEM((2, CHUNK, VOCAB), jnp.float32),
        pltpu.SemaphoreType.DMA,
        pltpu.SemaphoreType.DMA,
    ],
    compiler_params=pltpu.CompilerParams(use_tc_tiling_on_sc=False),
)
def _gather_kernel(ids_hbm, g_hbm, out_hbm, idx_v, rows_v, sem0, sem1):
    wid = lax.axis_index("s") * NUM_CORES + lax.axis_index("c")
    base = wid * ROWS_PER_TILE
    pltpu.sync_copy(ids_hbm.at[pl.ds(base, ROWS_PER_TILE)], idx_v)

    def pair_body(i, carry):
        j0 = 2 * i
        j1 = j0 + 1
        c0 = pltpu.async_copy(
            g_hbm.at[idx_v.at[pl.ds(j0 * CHUNK, CHUNK)]], rows_v.at[0], sem0)
        c1 = pltpu.async_copy(
            g_hbm.at[idx_v.at[pl.ds(j1 * CHUNK, CHUNK)]], rows_v.at[1], sem1)
        c0.wait()
        pltpu.sync_copy(rows_v.at[0], out_hbm.at[pl.ds(base + j0 * CHUNK, CHUNK)])
        c1.wait()
        pltpu.sync_copy(rows_v.at[1], out_hbm.at[pl.ds(base + j1 * CHUNK, CHUNK)])
        return carry

    lax.fori_loop(0, NCHUNK // 2, pair_body, 0)
    j = NCHUNK - 1
    c = pltpu.async_copy(
        g_hbm.at[idx_v.at[pl.ds(j * CHUNK, CHUNK)]], rows_v.at[0], sem0)
    c.wait()
    pltpu.sync_copy(rows_v.at[0], out_hbm.at[pl.ds(base + j * CHUNK, CHUNK)])


def kernel(input_ids, embed_table, proj_W):
    b, t = input_ids.shape
    ids = input_ids.reshape(-1).astype(jnp.int32)
    g = _compute_g(embed_table, proj_W)
    out = _gather_kernel(ids, g)
    return out.reshape(b, t, VOCAB)


# speedup vs baseline: 1.0851x; 1.0851x over previous
import functools

import jax
import jax.numpy as jnp
from jax import lax
from jax.experimental import pallas as pl
from jax.experimental.pallas import tpu as pltpu
from jax.experimental.pallas import tpu_sc as plsc

VOCAB = 1000
NUM_TOKENS = 1024 * 50

NUM_CORES = 2
NUM_SUBCORES = 16
NUM_WORKERS = NUM_CORES * NUM_SUBCORES

VC = 200
NKV = VOCAB // VC
NVT = VC // 8
NTB = 50 * 8
UNITS_PER_TILE = -(-NTB // NUM_WORKERS)


def _matmul_body(e_ref, p_ref, *g_refs):
    d = lax.dot_general(
        e_ref[...], p_ref[...],
        dimension_numbers=(((1,), (1,)), ((), ())),
        preferred_element_type=jnp.float32,
    )
    for k in range(NKV):
        g_refs[k][...] = d[:, k * VC:(k + 1) * VC]


def _compute_g(embed_table, proj_W):
    return pl.pallas_call(
        _matmul_body,
        out_shape=[jax.ShapeDtypeStruct((VOCAB, VC), jnp.float32)] * NKV,
    )(embed_table, proj_W)


def _transpose_chunk(gbuf, tbuf):
    iota = lax.iota(jnp.int32, 16)
    rowidx = [iota + 16 * c for c in range(8)]

    def vbody(vp, carry):
        vt = vp // 8
        vs = vp % 8
        col = jnp.full((16,), vp, jnp.int32)
        for c in range(8):
            vec = plsc.load_gather(gbuf, [rowidx[c], col])
            tbuf[vt, vs, pl.ds(c * 16, 16)] = vec
        return carry

    lax.fori_loop(0, VC, vbody, 0)


@functools.partial(
    pl.kernel,
    out_type=jax.ShapeDtypeStruct((50, 125, 8, 8, 128), jnp.float32),
    mesh=plsc.VectorSubcoreMesh(core_axis_name="c", subcore_axis_name="s"),
    scratch_types=[
        pltpu.VMEM((128,), jnp.int32),
        pltpu.VMEM((2, 128, VC), jnp.float32),
        pltpu.VMEM((2, NVT, 8, 128), jnp.float32),
        pltpu.SemaphoreType.DMA,
        pltpu.SemaphoreType.DMA,
        pltpu.SemaphoreType.DMA,
        pltpu.SemaphoreType.DMA,
    ],
    compiler_params=pltpu.CompilerParams(
        use_tc_tiling_on_sc=False, needs_layout_passes=False),
)
def _gather_kernel(idt_hbm, g0, g1, g2, g3, g4, out_hbm,
                   idx_v, gbuf, tbuf, gsem0, gsem1, wsem0, wsem1):
    gk = (g0, g1, g2, g3, g4)
    gsem = (gsem0, gsem1)
    wsem = (wsem0, wsem1)
    wid = lax.axis_index("s") * NUM_CORES + lax.axis_index("c")

    def unit(jj, carry):
        tb = wid + NUM_WORKERS * jj

        @pl.when(tb < NTB)
        def _():
            t = tb // 8
            bb = tb % 8
            pltpu.sync_copy(idt_hbm.at[pl.ds(tb * 128, 128)], idx_v)
            gd = [None] * NKV
            wd = [None] * NKV
            gd[0] = pltpu.async_copy(gk[0].at[idx_v], gbuf.at[0], gsem[0])
            for kv in range(NKV):
                p = kv & 1
                if kv + 1 < NKV:
                    gd[kv + 1] = pltpu.async_copy(
                        gk[kv + 1].at[idx_v], gbuf.at[1 - p], gsem[1 - p])
                gd[kv].wait()
                if kv >= 2:
                    wd[kv - 2].wait()
                _transpose_chunk(gbuf.at[p], tbuf.at[p])
                wd[kv] = pltpu.async_copy(
                    tbuf.at[p], out_hbm.at[t, pl.ds(kv * NVT, NVT), bb],
                    wsem[p])
            wd[NKV - 2].wait()
            wd[NKV - 1].wait()

        return carry

    lax.fori_loop(0, UNITS_PER_TILE, unit, 0)


def kernel(input_ids, embed_table, proj_W):
    b, t = input_ids.shape
    ids_t = input_ids.T.reshape(-1).astype(jnp.int32)
    gs = _compute_g(embed_table, proj_W)
    out5 = _gather_kernel(ids_t, *gs)
    return out5.transpose(2, 4, 0, 1, 3).reshape(b, t, VOCAB)
